# Initial kernel scaffold; baseline (speedup 1.0000x reference)
import jax
import jax.numpy as jnp
from jax.experimental import pallas as pl


def kernel(x, edge_index, edge_attr, W_edge, b_edge, W_l, b_l, W_r):
    raise NotImplementedError("write your pallas kernel here")


# speedup vs baseline: 2.5331x; 2.5331x over previous
import functools

import jax
import jax.numpy as jnp
from jax import lax
from jax.experimental import pallas as pl
from jax.experimental.pallas import tpu as pltpu
from jax.experimental.pallas import tpu_sc as plsc

NC = 2
NS = 16
L = 16
CHUNK = 128
RING = 16
AW = 32


def _sc_aggregate(N, D, CPT, RPT):
    NPAD = NS * RPT
    NB = CPT // RING
    mesh = plsc.VectorSubcoreMesh(core_axis_name="c", subcore_axis_name="s",
                                  num_cores=NC, num_subcores=NS)
    out_type = (
        jax.ShapeDtypeStruct((NC, NPAD, D), jnp.float32),
        jax.ShapeDtypeStruct((NC, NPAD, AW), jnp.float32),
    )
    scratch = [
        pltpu.VMEM_SHARED((NPAD, D), jnp.float32),
        pltpu.VMEM_SHARED((NPAD, AW), jnp.float32),
        pltpu.VMEM((RING, CHUNK), jnp.int32),
        pltpu.VMEM((RING, CHUNK), jnp.int32),
        pltpu.VMEM((CHUNK, AW), jnp.float32),
        pltpu.VMEM((CHUNK, D), jnp.float32),
        pltpu.SemaphoreType.DMA,
    ]
    offs = sorted(set(list(range(0, RPT - CHUNK, CHUNK)) + [RPT - CHUNK]))

    @functools.partial(pl.kernel, out_type=out_type, mesh=mesh,
                       scratch_types=scratch,
                       compiler_params=pltpu.CompilerParams(
                           use_tc_tiling_on_sc=False))
    def agg(x_hbm, src_hbm, dst_hbm, aug_hbm, sx_out, sa_out,
            sx_sh, sa_sh, src_v, dst_v, aug_v, gbuf, sem):
        c = lax.axis_index("c")
        s = lax.axis_index("s")
        w = s * NC + c

        zeros = jnp.zeros((L,), jnp.float32)

        def zrow(i, carry):
            def zcol(j, carry2):
                gbuf[i, pl.ds(j * L, L)] = zeros
                return carry2
            lax.fori_loop(0, D // L, zcol, 0)
            aug_v[i, pl.ds(0, L)] = zeros
            aug_v[i, pl.ds(L, L)] = zeros
            return carry
        lax.fori_loop(0, CHUNK, zrow, 0)

        row0 = s * RPT
        for off in offs:
            pltpu.sync_copy(gbuf, sx_sh.at[pl.ds(row0 + off, CHUNK)])
            pltpu.sync_copy(aug_v, sa_sh.at[pl.ds(row0 + off, CHUNK)])
        plsc.subcore_barrier()

        def block(b, carry):
            pltpu.sync_copy(src_hbm.at[w, pl.ds(b * RING, RING)], src_v)
            pltpu.sync_copy(dst_hbm.at[w, pl.ds(b * RING, RING)], dst_v)

            def body(j, carry2):
                g = b * RING + j
                pltpu.sync_copy(aug_hbm.at[w, g], aug_v)
                pltpu.async_copy(x_hbm.at[src_v.at[j]], gbuf, sem).wait()
                didx = dst_v.at[j]
                pltpu.sync_copy(gbuf, sx_sh.at[didx], add=True)
                pltpu.sync_copy(aug_v, sa_sh.at[didx], add=True)
                return carry2
            lax.fori_loop(0, RING, body, 0)
            return carry
        lax.fori_loop(0, NB, block, 0)
        plsc.subcore_barrier()

        for off in offs:
            r = row0 + off
            pltpu.sync_copy(sx_sh.at[pl.ds(r, CHUNK)], gbuf)
            pltpu.sync_copy(gbuf, sx_out.at[c, pl.ds(r, CHUNK)])
            pltpu.sync_copy(sa_sh.at[pl.ds(r, CHUNK)], aug_v)
            pltpu.sync_copy(aug_v, sa_out.at[c, pl.ds(r, CHUNK)])

    return agg


def _tc_combine(N, D, ED, BT=512):
    def body(sx_ref, sa_ref, x_ref, We_ref, be_ref, Wl_ref, bl_ref,
             Wr_ref, out_ref):
        sx = sx_ref[0] + sx_ref[1]
        sa = sa_ref[0] + sa_ref[1]
        se = sa[:, :ED]
        cnt = sa[:, ED:ED + 1]
        num = (sx + jnp.dot(se, We_ref[...], preferred_element_type=jnp.float32)
               + cnt * be_ref[...])
        agg = num / jnp.maximum(cnt, 1.0)
        out_ref[...] = (
            jnp.dot(agg, Wl_ref[...], preferred_element_type=jnp.float32)
            + bl_ref[...]
            + jnp.dot(x_ref[...], Wr_ref[...], preferred_element_type=jnp.float32))

    return pl.pallas_call(
        body,
        grid=(pl.cdiv(N, BT),),
        in_specs=[
            pl.BlockSpec((NC, BT, D), lambda i: (0, i, 0)),
            pl.BlockSpec((NC, BT, AW), lambda i: (0, i, 0)),
            pl.BlockSpec((BT, D), lambda i: (i, 0)),
            pl.BlockSpec((ED, D), lambda i: (0, 0)),
            pl.BlockSpec((1, D), lambda i: (0, 0)),
            pl.BlockSpec((D, D), lambda i: (0, 0)),
            pl.BlockSpec((1, D), lambda i: (0, 0)),
            pl.BlockSpec((D, D), lambda i: (0, 0)),
        ],
        out_specs=pl.BlockSpec((BT, D), lambda i: (i, 0)),
        out_shape=jax.ShapeDtypeStruct((N, D), jnp.float32),
    )


def kernel(x, edge_index, edge_attr, W_edge, b_edge, W_l, b_l, W_r):
    N, D = x.shape
    E = edge_index.shape[1]
    ED = edge_attr.shape[1]
    NW = NC * NS
    CPT = -(-E // (NW * CHUNK))
    CPT = -(-CPT // RING) * RING
    EPAD = NW * CPT * CHUNK
    RPT = max(-(-(N + 1) // NS), CHUNK)
    RPT = -(-RPT // 8) * 8

    x = x.astype(jnp.float32)
    src = edge_index[0].astype(jnp.int32)
    dst = edge_index[1].astype(jnp.int32)
    pad = EPAD - E
    src_p = jnp.concatenate([src, jnp.zeros((pad,), jnp.int32)]).reshape(
        NW, CPT, CHUNK)
    dst_p = jnp.concatenate([dst, jnp.full((pad,), N, jnp.int32)]).reshape(
        NW, CPT, CHUNK)
    aug = jnp.concatenate(
        [edge_attr.astype(jnp.float32),
         jnp.ones((E, 1), jnp.float32),
         jnp.zeros((E, AW - ED - 1), jnp.float32)], axis=1)
    aug_p = jnp.concatenate([aug, jnp.zeros((pad, AW), jnp.float32)]).reshape(
        NW, CPT, CHUNK, AW)

    sxp, sap = _sc_aggregate(N, D, CPT, RPT)(x, src_p, dst_p, aug_p)
    out = _tc_combine(N, D, ED)(
        sxp, sap, x, W_edge.astype(jnp.float32),
        b_edge.astype(jnp.float32).reshape(1, D), W_l.astype(jnp.float32),
        b_l.astype(jnp.float32).reshape(1, D), W_r.astype(jnp.float32))
    return out
